# baseline (speedup 1.0000x reference)
import jax
import jax.numpy as jnp
from jax.experimental import pallas as pl
from jax.experimental.pallas import tpu as pltpu

B, G, NGO, NKE, NN, GRID, NT, DC = 32, 10000, 2000, 500, 2, 2, 50, 256
F32 = jnp.float32
MBLK = 200


def _main_body(gp_ref, adj_ref, wspg_ref, prm_ref, h_ref, sp_ref, zt_ref):
    i = pl.program_id(0)

    @pl.when(i == 0)
    def _build_zt():
        g = gp_ref[:, 0:32]
        gamma = gp_ref[:, 32:33]
        beta = gp_ref[:, 33:34]
        t0 = jnp.tanh(g * prm_ref[0, 0] + prm_ref[0, 2])
        t1 = jnp.tanh(g * prm_ref[0, 1] + prm_ref[0, 3])
        s1 = jnp.sum(t0, axis=1, keepdims=True) + jnp.sum(t1, axis=1, keepdims=True)
        s2 = jnp.sum(t0 * t0, axis=1, keepdims=True) + jnp.sum(t1 * t1, axis=1, keepdims=True)
        mean = s1 * (1.0 / (2 * B))
        var = s2 * (1.0 / (2 * B)) - mean * mean
        inv = jax.lax.rsqrt(var + 1e-5) * gamma
        zt_ref[...] = jnp.zeros((G, 128), F32)
        zt_ref[:, 0:32] = (t0 - mean) * inv + beta
        zt_ref[:, 32:64] = (t1 - mean) * inv + beta
        zt_ref[:, 64:65] = jnp.ones((G, 1), F32)
        sp_ref[...] = jnp.dot(wspg_ref[...], zt_ref[...], preferred_element_type=F32)

    h_ref[...] = jnp.dot(adj_ref[...], zt_ref[...], preferred_element_type=F32)


def _fk2(x0, x1, cref):
    def c(ci, j, i, g):
        return cref[0, ci * 8 + j * 4 + i * 2 + g]
    c10, c20 = jnp.cos(x0), jnp.cos(2.0 * x0)
    s10, s20 = jnp.sin(x0), jnp.sin(2.0 * x0)
    c11, c21 = jnp.cos(x1), jnp.cos(2.0 * x1)
    s11, s21 = jnp.sin(x1), jnp.sin(2.0 * x1)
    y0 = (c10 * c(0, 0, 0, 0) + c20 * c(0, 0, 0, 1) + c11 * c(0, 0, 1, 0)
          + c21 * c(0, 0, 1, 1) + s10 * c(1, 0, 0, 0) + s20 * c(1, 0, 0, 1)
          + s11 * c(1, 0, 1, 0) + s21 * c(1, 0, 1, 1))
    y1 = (c10 * c(0, 1, 0, 0) + c20 * c(0, 1, 0, 1) + c11 * c(0, 1, 1, 0)
          + c21 * c(0, 1, 1, 1) + s10 * c(1, 1, 0, 0) + s20 * c(1, 1, 0, 1)
          + s11 * c(1, 1, 1, 0) + s21 * c(1, 1, 1, 1))
    return y0, y1


def _fourier32(x, p, bias):
    j = p.shape[1] // 4
    y = (jnp.dot(jnp.cos(x), p[:, 0:j], preferred_element_type=F32)
         + jnp.dot(jnp.cos(2.0 * x), p[:, j:2 * j], preferred_element_type=F32)
         + jnp.dot(jnp.sin(x), p[:, 2 * j:3 * j], preferred_element_type=F32)
         + jnp.dot(jnp.sin(2.0 * x), p[:, 3 * j:4 * j], preferred_element_type=F32))
    return y + bias


def _tail_body(h_ref, sprow_ref, goke_ref, keke_ref, nb_g2g_ref, nb_g2k_ref,
               nb_k0_ref, nb_k1_ref, comp_ref, wdrugT_ref, wbioT_ref, tis_ref,
               bio1p_ref, drug1p_ref, predp_ref, bio1b_ref, drug1b_ref,
               bbio_ref, bdrug_ref, wspgo_ref, wspke_ref, c_g2g_ref,
               c_g2k_ref, c_k0_ref, c_k1_ref, sc_ref,
               pred_ref, state_ref):
    sc = lambda k: sc_ref[0, k]

    deg = h_ref[:, 64:65] + 1e-8
    x0 = h_ref[:, 0:32] / deg
    x1 = h_ref[:, 32:64] / deg
    y0, y1 = _fk2(x0, x1, c_g2g_ref)
    y0 = y0 + nb_g2g_ref[:, 0:1]
    y1 = y1 + nb_g2g_ref[:, 1:2]
    e = y0 * sc(0) + y1 * sc(1) + sc(2)
    g0 = jnp.tanh(e * sc(3) + sc(5))
    g1 = jnp.tanh(e * sc(4) + sc(6))
    gost = g0 * sc(7) + g1 * sc(8)
    sp_go = (jnp.dot(wspgo_ref[...], gost, preferred_element_type=F32)
             + sc(9) * jnp.sum(wspgo_ref[...]))

    a = goke_ref[...]
    dk = jnp.sum(a, axis=1, keepdims=True) + 1e-8
    x0 = jnp.dot(a, g0, preferred_element_type=F32) / dk
    x1 = jnp.dot(a, g1, preferred_element_type=F32) / dk
    y0, y1 = _fk2(x0, x1, c_g2k_ref)
    k0 = y0 + nb_g2k_ref[:, 0:1]
    k1 = y1 + nb_g2k_ref[:, 1:2]

    a = keke_ref[...]
    dk = jnp.sum(a, axis=1, keepdims=True) + 1e-8
    x0 = jnp.dot(a, k0, preferred_element_type=F32) / dk
    x1 = jnp.dot(a, k1, preferred_element_type=F32) / dk
    y0, y1 = _fk2(x0, x1, c_k0_ref)
    k0 = y0 + nb_k0_ref[:, 0:1]
    k1 = y1 + nb_k0_ref[:, 1:2]
    x0 = jnp.dot(a, k0, preferred_element_type=F32) / dk
    x1 = jnp.dot(a, k1, preferred_element_type=F32) / dk
    y0, y1 = _fk2(x0, x1, c_k1_ref)
    k0 = y0 + nb_k1_ref[:, 0:1]
    k1 = y1 + nb_k1_ref[:, 1:2]

    kest = k0 * sc(10) + k1 * sc(11)
    sp_ke = (jnp.dot(wspke_ref[...], kest, preferred_element_type=F32)
             + sc(12) * jnp.sum(wspke_ref[...]))
    sp_gene = (sprow_ref[:, 0:32] * sc(16) + sprow_ref[:, 32:64] * sc(17)
               + sprow_ref[:, 64:65] * sc(18))
    state_ref[...] = sp_gene + sp_go + sp_ke + sc(19)

    kelay = k0 * sc(13) + k1 * sc(14) + sc(15)
    kidx = jax.lax.broadcasted_iota(jnp.int32, (NKE, 64), 0)
    m = (kidx == tis_ref[...]).astype(F32)
    bio_bt = jax.lax.dot_general(kelay, m, (((0,), (0,)), ((), ())),
                                 preferred_element_type=F32)
    xb = jnp.dot(bio_bt, wbioT_ref[...], preferred_element_type=F32) + bbio_ref[...]
    yb = _fourier32(xb, bio1p_ref[...], bio1b_ref[...])

    xd = jnp.dot(comp_ref[...], wdrugT_ref[...], preferred_element_type=F32) + bdrug_ref[...]
    yd = _fourier32(xd, drug1p_ref[...], drug1b_ref[...])

    comb = jnp.concatenate([yb, yd], axis=1)
    yp = _fourier32(comb, predp_ref[...], jnp.zeros((1, 1), F32)) + sc(20)
    pred_ref[...] = yp


def _packT(coeffs):
    return jnp.concatenate([coeffs[0, :, :, 0].T, coeffs[0, :, :, 1].T,
                            coeffs[1, :, :, 0].T, coeffs[1, :, :, 1].T], axis=1)


def kernel(gene, gene_go, go_ke, ke_ke, tissue, compound, W_gene1, b_gene1,
           bn_gamma, bn_beta, W_gstate, b_gstate, g2g_coeffs, g2g_nbias,
           W_goenc, b_goenc, W_godec, b_godec, W_gostate, b_gostate,
           g2k_coeffs, g2k_nbias, k2k0_coeffs, k2k0_nbias, k2k1_coeffs,
           k2k1_nbias, W_kestate, b_kestate, W_kelayer, b_kelayer, W_sp, b_sp,
           W_bio0, b_bio0, bio1_coeffs, bio1_bias, W_drug0, b_drug0,
           drug1_coeffs, drug1_bias, pred_coeffs, pred_bias):
    gene_pack = jnp.concatenate(
        [gene.T, bn_gamma[:, None], bn_beta[:, None]], axis=1)
    prm = jnp.stack([W_gene1[0, 0], W_gene1[1, 0], b_gene1[0], b_gene1[1]]
                    ).reshape(1, 4)
    wspg = W_sp[:, :G]

    h, sprow = pl.pallas_call(
        _main_body,
        grid=(NGO // MBLK,),
        in_specs=[
            pl.BlockSpec((G, 34), lambda i: (0, 0)),
            pl.BlockSpec((MBLK, G), lambda i: (i, 0)),
            pl.BlockSpec((1, G), lambda i: (0, 0)),
            pl.BlockSpec(memory_space=pltpu.SMEM),
        ],
        out_specs=[
            pl.BlockSpec((MBLK, 128), lambda i: (i, 0)),
            pl.BlockSpec((1, 128), lambda i: (0, 0)),
        ],
        out_shape=[
            jax.ShapeDtypeStruct((NGO, 128), F32),
            jax.ShapeDtypeStruct((1, 128), F32),
        ],
        scratch_shapes=[pltpu.VMEM((G, 128), F32)],
    )(gene_pack, gene_go, wspg, prm)

    tis = jnp.full((1, 64), -1, jnp.int32).at[0, :NT].set(tissue.astype(jnp.int32))
    wbioT = jnp.zeros((64, B), F32).at[:NT, :].set(W_bio0.T)
    scal = jnp.stack([
        W_goenc[0, 0], W_goenc[0, 1], b_goenc[0],
        W_godec[0, 0], W_godec[1, 0], b_godec[0], b_godec[1],
        W_gostate[0, 0], W_gostate[0, 1], b_gostate[0],
        W_kestate[0, 0], W_kestate[0, 1], b_kestate[0],
        W_kelayer[0, 0], W_kelayer[0, 1], b_kelayer[0],
        W_gstate[0, 0], W_gstate[0, 1], b_gstate[0],
        b_sp[0], pred_bias[0, 0], 0.0, 0.0, 0.0]).reshape(1, 24)

    vm = pl.BlockSpec(memory_space=pltpu.VMEM)
    sm = pl.BlockSpec(memory_space=pltpu.SMEM)
    pred, state_row = pl.pallas_call(
        _tail_body,
        in_specs=[vm] * 21 + [sm] * 5,
        out_specs=[vm, vm],
        out_shape=[
            jax.ShapeDtypeStruct((B, 1), F32),
            jax.ShapeDtypeStruct((1, B), F32),
        ],
    )(h, sprow, go_ke, ke_ke, g2g_nbias, g2k_nbias, k2k0_nbias, k2k1_nbias,
      compound, W_drug0.T, wbioT, tis, _packT(bio1_coeffs),
      _packT(drug1_coeffs), _packT(pred_coeffs), bio1_bias, drug1_bias,
      b_bio0.reshape(1, B), b_drug0.reshape(1, B),
      W_sp[:, G:G + NGO], W_sp[:, G + NGO:],
      g2g_coeffs.reshape(1, 16), g2k_coeffs.reshape(1, 16),
      k2k0_coeffs.reshape(1, 16), k2k1_coeffs.reshape(1, 16), scal)

    return pred, state_row.reshape(B, 1)
